# Initial kernel scaffold; baseline (speedup 1.0000x reference)
import jax
import jax.numpy as jnp
from jax.experimental import pallas as pl


def kernel(node_feats, edge_feats, fp_vector, edge_index, node_graph_ids, W_mpnn, b_mpnn, W_edge, b_edge, W_fp, b_fp, bn_gamma, bn_beta, bn_mean, bn_var, W1, b1, W2, b2, W3, b3):
    raise NotImplementedError("write your pallas kernel here")


# speedup vs baseline: 6.7328x; 6.7328x over previous
import functools

import jax
import jax.numpy as jnp
from jax import lax
from jax.experimental import pallas as pl
from jax.experimental.pallas import tpu as pltpu
from jax.experimental.pallas import tpu_sc as plsc

N_NODES = 10000
N_EDGES = 320000
D_FEAT = 128
D_EDGE = 16
NODE_OUT = 128
EDGE_OUT = 16
FP_DIM = 2048
FP_EMBED = 128
BATCH = 64

NC = 2
NS = 16
NW = NC * NS

EW = 10240
E_PAD = NW * EW
CHUNK = 1024
NCHUNK = EW // CHUNK
NPAD = 10240
HSIZE = BATCH * NPAD
ZW = 4096
ZPT = HSIZE // (NS * ZW)


def _sc_body(src_hbm, dst_hbm, ngid_hbm, ef_hbm, hist_out, feat_out,
             hist_sh, ngid_v, src_v, dst_v, gd_v, ef_v, idx_v, ones_v,
             acc_v, zero_v):
  c = lax.axis_index("c")
  s = lax.axis_index("s")
  wid = c * NS + s

  pltpu.sync_copy(ngid_hbm, ngid_v)

  zeros16 = jnp.zeros((16,), jnp.float32)
  ones16 = jnp.ones((16,), jnp.float32)
  for r in range(8):
    for cc in range(8):
      ones_v[r, pl.ds(cc * 16, 16)] = ones16
  for g in range(BATCH):
    acc_v[g, :] = zeros16

  def _zb(i, _):
    zero_v[pl.ds(i * 16, 16)] = zeros16
    return 0
  lax.fori_loop(0, ZW // 16, _zb, 0)

  for k in range(ZPT):
    pltpu.sync_copy(zero_v, hist_sh.at[pl.ds((s * ZPT + k) * ZW, ZW)])
  plsc.subcore_barrier()

  def _chunk(k, _):
    base = wid * EW + k * CHUNK
    pltpu.sync_copy(src_hbm.at[pl.ds(base, CHUNK)], src_v)
    pltpu.sync_copy(dst_hbm.at[pl.ds(base, CHUNK)], dst_v)
    pltpu.sync_copy(ef_hbm.at[pl.ds(base * 16, CHUNK * 16)], ef_v)
    for r in range(8):
      for cc in range(8):
        off = r * 128 + cc * 16
        d16 = dst_v[pl.ds(off, 16)]
        g16 = plsc.load_gather(ngid_v, [d16])
        gd_v[pl.ds(off, 16)] = g16
        s16 = src_v[pl.ds(off, 16)]
        idx_v[r, pl.ds(cc * 16, 16)] = g16 * NPAD + s16
      pltpu.sync_copy(ones_v.at[r], hist_sh.at[idx_v.at[r]], add=True)

    def _fe(e, _):
      g16 = gd_v[pl.ds(e * 16, 16)]
      for u in range(16):
        ee = e * 16 + u
        plsc.addupdate(acc_v.at[g16[u]], ef_v[pl.ds(ee * 16, 16)])
      return 0
    lax.fori_loop(0, CHUNK // 16, _fe, 0)
    return 0

  lax.fori_loop(0, NCHUNK, _chunk, 0)

  plsc.subcore_barrier()
  for k in range(ZPT):
    off = (s * ZPT + k) * ZW
    pltpu.sync_copy(hist_sh.at[pl.ds(off, ZW)], hist_out.at[c, pl.ds(off, ZW)])
  pltpu.sync_copy(acc_v, feat_out.at[wid])


@functools.cache
def _sc_edge_agg():
  return functools.partial(
      pl.kernel,
      out_type=(
          jax.ShapeDtypeStruct((NC, HSIZE), jnp.float32),
          jax.ShapeDtypeStruct((NW, BATCH, D_EDGE), jnp.float32),
      ),
      mesh=plsc.VectorSubcoreMesh(
          core_axis_name="c", subcore_axis_name="s",
          num_cores=NC, num_subcores=NS),
      compiler_params=pltpu.CompilerParams(needs_layout_passes=False),
      scratch_types=[
        pltpu.VMEM_SHARED((HSIZE,), jnp.float32),
        pltpu.VMEM((N_NODES,), jnp.int32),
        pltpu.VMEM((CHUNK,), jnp.int32),
        pltpu.VMEM((CHUNK,), jnp.int32),
        pltpu.VMEM((CHUNK,), jnp.int32),
        pltpu.VMEM((CHUNK * 16,), jnp.float32),
        pltpu.VMEM((8, 128), jnp.int32),
        pltpu.VMEM((8, 128), jnp.float32),
          pltpu.VMEM((BATCH, D_EDGE), jnp.float32),
          pltpu.VMEM((ZW,), jnp.float32),
      ],
  )(_sc_body)


def _dot(a, b, precision=None):
  return lax.dot_general(a, b, (((1,), (0,)), ((), ())),
                         precision=precision,
                         preferred_element_type=jnp.float32)


def _tc_body(hist_ref, feat_ref, nf_ref, wm_ref, bm_ref, we_ref, be_ref,
             fp_ref, wfp_ref, bfp_ref, gam_ref, bet_ref, mu_ref, var_ref,
             w1_ref, b1_ref, w2_ref, b2_ref, w3_ref, b3_ref, out_ref):
  S = hist_ref[0] + hist_ref[1]
  col = lax.broadcasted_iota(jnp.int32, (BATCH, NPAD), 1)
  S = jnp.where(col == N_NODES, 0.0, S)
  cnt = jnp.sum(S, axis=1, keepdims=True)

  node_enc = jnp.maximum(_dot(nf_ref[...], wm_ref[...]) + bm_ref[...], 0.0)
  node_enc = jnp.concatenate(
      [node_enc, jnp.zeros((NPAD - N_NODES, NODE_OUT), jnp.float32)], axis=0)
  node_part = _dot(S, node_enc, lax.Precision.HIGHEST)

  esum = jnp.sum(feat_ref[...], axis=0)
  edge_part = _dot(esum, we_ref[...]) + cnt * be_ref[...]

  m = jnp.maximum(jnp.max(node_part, axis=1, keepdims=True),
                  jnp.max(edge_part, axis=1, keepdims=True))
  en = jnp.exp(node_part - m)
  ee = jnp.exp(edge_part - m)
  z = jnp.sum(en, axis=1, keepdims=True) + jnp.sum(ee, axis=1, keepdims=True)
  mol_n = en / z
  mol_e = ee / z

  h = _dot(fp_ref[...], wfp_ref[...]) + bfp_ref[...]
  h = (h - mu_ref[...]) * lax.rsqrt(var_ref[...] + 1e-5) * gam_ref[...] \
      + bet_ref[...]
  fpe = jnp.maximum(h, 0.0)

  h1 = (_dot(mol_n, w1_ref[0:NODE_OUT, :])
        + _dot(mol_e, w1_ref[NODE_OUT:NODE_OUT + EDGE_OUT, :])
        + _dot(fpe, w1_ref[NODE_OUT + EDGE_OUT:, :])
        + b1_ref[...])
  h1 = jnp.maximum(h1, 0.0)
  h2 = jnp.maximum(_dot(h1, w2_ref[...]) + b2_ref[...], 0.0)
  out_ref[...] = _dot(h2, w3_ref[...]) + b3_ref[...]


_tc_tail = pl.pallas_call(
    _tc_body,
    out_shape=jax.ShapeDtypeStruct((BATCH, NODE_OUT), jnp.float32),
)


def kernel(node_feats, edge_feats, fp_vector, edge_index, node_graph_ids,
           W_mpnn, b_mpnn, W_edge, b_edge, W_fp, b_fp, bn_gamma, bn_beta,
           bn_mean, bn_var, W1, b1, W2, b2, W3, b3):
  pad = E_PAD - N_EDGES
  src = edge_index[0]
  dst = edge_index[1]
  src_p = jnp.concatenate([src, jnp.full((pad,), N_NODES, jnp.int32)])
  dst_p = jnp.concatenate([dst, jnp.zeros((pad,), jnp.int32)])
  ef_p = jnp.concatenate(
      [edge_feats, jnp.zeros((pad, D_EDGE), jnp.float32)]).reshape(-1)

  hist, feat = _sc_edge_agg()(src_p, dst_p, node_graph_ids, ef_p)
  hist = hist.reshape(NC, BATCH, NPAD)

  out = _tc_tail(
      hist, feat, node_feats, W_mpnn, b_mpnn.reshape(1, -1), W_edge,
      b_edge.reshape(1, -1), fp_vector, W_fp, b_fp.reshape(1, -1),
      bn_gamma.reshape(1, -1), bn_beta.reshape(1, -1),
      bn_mean.reshape(1, -1), bn_var.reshape(1, -1), W1,
      b1.reshape(1, -1), W2, b2.reshape(1, -1), W3, b3.reshape(1, -1))
  return out
